# baseline (speedup 1.0000x reference)
import jax
import jax.numpy as jnp
from jax import lax
from jax.experimental import pallas as pl
from jax.experimental.pallas import tpu as pltpu
from jax.experimental.pallas import tpu_sc as plsc

B, S, H = 4, 2048, 1024
E, K, FF = 8, 2, 4096
N = B * S
NK = N * K

T = 256
NB = NK // T + E
NP = NB * T

TB = 512
NT = N // TB

NEG_BIG = -1e30

NC = 2
NS = 16
NW = NC * NS
CH = 32


def _gating_body(x_ref, wg_ref, bg_ref, eidx_ref, gates_ref, ranks_ref,
                 colsum_ref, counts_ref, carry_ref):
    t = pl.program_id(0)

    @pl.when(t == 0)
    def _():
        carry_ref[...] = jnp.zeros_like(carry_ref)

    logits = jnp.dot(x_ref[...], wg_ref[...],
                     preferred_element_type=jnp.float32) + bg_ref[...]
    m = jnp.max(logits, axis=1, keepdims=True)
    ex = jnp.exp(logits - m)
    probs = ex / jnp.sum(ex, axis=1, keepdims=True)
    colsum_ref[0] = jnp.sum(probs, axis=0, keepdims=True)

    iota = lax.broadcasted_iota(jnp.int32, (1, E), 1)
    p1 = jnp.max(probs, axis=1, keepdims=True)
    i1 = jnp.min(jnp.where(probs == p1, iota, E), axis=1, keepdims=True)
    masked = jnp.where(iota == i1, NEG_BIG, probs)
    p2 = jnp.max(masked, axis=1, keepdims=True)
    i2 = jnp.min(jnp.where(masked == p2, iota, E), axis=1, keepdims=True)
    dsum = p1 + p2
    gates_ref[...] = jnp.concatenate([p1 / dsum, p2 / dsum], axis=1)
    eidx_ref[...] = jnp.concatenate([i1, i2], axis=1)

    onehot = (iota == i1).astype(jnp.float32) + (iota == i2).astype(jnp.float32)
    r_io = lax.broadcasted_iota(jnp.int32, (TB, TB), 0)
    c_io = lax.broadcasted_iota(jnp.int32, (TB, TB), 1)
    tril = (r_io > c_io).astype(jnp.float32)
    prefix = jnp.dot(tril, onehot, preferred_element_type=jnp.float32)
    base = prefix + carry_ref[...]
    r1 = jnp.sum(jnp.where(iota == i1, base, 0.0), axis=1, keepdims=True)
    r2 = jnp.sum(jnp.where(iota == i2, base, 0.0), axis=1, keepdims=True)
    ranks_ref[...] = jnp.concatenate([r1, r2], axis=1).astype(jnp.int32)

    new_carry = carry_ref[...] + jnp.sum(onehot, axis=0, keepdims=True)
    carry_ref[...] = new_carry
    counts_ref[...] = new_carry


def _make_gather_body(rows_per_worker, src_rows, dtype):
    G = rows_per_worker // CH
    assert G % 2 == 0 and rows_per_worker % CH == 0

    def body(idx_hbm, src_hbm, out_hbm, idx_all, buf_a, buf_b,
             sg_a, sg_b, sw_a, sw_b):
        wid = lax.axis_index("s") * NC + lax.axis_index("c")
        base = wid * rows_per_worker
        pltpu.sync_copy(idx_hbm.at[pl.ds(base, rows_per_worker)], idx_all)

        def gather(g, buf, sem):
            return pltpu.async_copy(
                src_hbm.at[idx_all.at[pl.ds(g * CH, CH)]], buf, sem)

        def write(g, buf, sem):
            return pltpu.async_copy(
                buf, out_hbm.at[pl.ds(base + g * CH, CH)], sem)

        def wait_gather(buf, sem):
            pltpu.make_async_copy(src_hbm.at[pl.ds(0, CH)], buf, sem).wait()

        def wait_write(g, buf, sem):
            pltpu.make_async_copy(
                buf, out_hbm.at[pl.ds(base + g * CH, CH)], sem).wait()

        gather(0, buf_a, sg_a)

        def step(j, carry):
            g0 = 2 * j
            wait_gather(buf_a, sg_a)
            gather(g0 + 1, buf_b, sg_b)
            write(g0, buf_a, sw_a)
            wait_gather(buf_b, sg_b)
            wait_write(g0, buf_a, sw_a)
            g_next = jnp.minimum(g0 + 2, G - 1)
            gather(g_next, buf_a, sg_a)
            write(g0 + 1, buf_b, sw_b)
            wait_write(g0 + 1, buf_b, sw_b)
            return carry

        lax.fori_loop(0, G // 2, step, 0)
        wait_gather(buf_a, sg_a)

    scratch = [
        pltpu.VMEM((rows_per_worker,), jnp.int32),
        pltpu.VMEM((CH, H), dtype),
        pltpu.VMEM((CH, H), dtype),
        pltpu.SemaphoreType.DMA,
        pltpu.SemaphoreType.DMA,
        pltpu.SemaphoreType.DMA,
        pltpu.SemaphoreType.DMA,
    ]
    return body, scratch


def _sc_gather(idx, src, n_rows, dtype):
    body, scratch = _make_gather_body(n_rows // NW, src.shape[0], dtype)
    return pl.kernel(
        body,
        out_type=jax.ShapeDtypeStruct((n_rows, H), dtype),
        mesh=plsc.VectorSubcoreMesh(
            core_axis_name="c", subcore_axis_name="s",
            num_cores=NC, num_subcores=NS),
        scratch_types=scratch,
    )(idx, src)


def _ffn_body(be_ref, valid_ref, xg_ref, w1_ref, w2_ref, b1_ref, b2_ref,
              out_ref):
    b = pl.program_id(0)

    @pl.when(valid_ref[b] == 1)
    def _():
        h = jnp.dot(xg_ref[...], w1_ref[0],
                    preferred_element_type=jnp.float32)
        h = jnp.maximum(h + b1_ref[0, 0], 0.0).astype(jnp.bfloat16)
        y = jnp.dot(h, w2_ref[0], preferred_element_type=jnp.float32)
        out_ref[...] = y + b2_ref[0]

    @pl.when(valid_ref[b] == 0)
    def _():
        out_ref[...] = jnp.zeros_like(out_ref)


def _combine_body(a_ref, b_ref, g_ref, o_ref):
    g = g_ref[...]
    o_ref[...] = g[:, 0:1] * a_ref[...] + g[:, 1:2] * b_ref[...]


def kernel(x, Wg, bg, W1, b1, W2, b2):
    x_flat = x.reshape(-1, H)

    eidx, gates2, ranks, colsum, counts = pl.pallas_call(
        _gating_body,
        grid=(NT,),
        in_specs=[
            pl.BlockSpec((TB, H), lambda t: (t, 0)),
            pl.BlockSpec((H, E), lambda t: (0, 0)),
            pl.BlockSpec((1, E), lambda t: (0, 0)),
        ],
        out_specs=[
            pl.BlockSpec((TB, K), lambda t: (t, 0)),
            pl.BlockSpec((TB, K), lambda t: (t, 0)),
            pl.BlockSpec((TB, K), lambda t: (t, 0)),
            pl.BlockSpec((1, 1, E), lambda t: (t, 0, 0)),
            pl.BlockSpec((1, E), lambda t: (0, 0)),
        ],
        out_shape=[
            jax.ShapeDtypeStruct((N, K), jnp.int32),
            jax.ShapeDtypeStruct((N, K), jnp.float32),
            jax.ShapeDtypeStruct((N, K), jnp.int32),
            jax.ShapeDtypeStruct((NT, 1, E), jnp.float32),
            jax.ShapeDtypeStruct((1, E), jnp.float32),
        ],
        scratch_shapes=[pltpu.VMEM((1, E), jnp.float32)],
    )(x_flat, Wg, bg.reshape(1, E))

    cs = jnp.sum(colsum, axis=(0, 1))
    aux = E * jnp.sum((cs / jnp.sum(cs)) * (cs / N))

    counts_i = counts[0].astype(jnp.int32)
    blocks_e = (counts_i + (T - 1)) // T
    cumb = jnp.concatenate(
        [jnp.zeros((1,), jnp.int32), jnp.cumsum(blocks_e, dtype=jnp.int32)])
    bar = jnp.arange(NB, dtype=jnp.int32)
    block_expert = jnp.sum(
        (bar[:, None] >= cumb[None, 1:E]).astype(jnp.int32), axis=1)
    valid = (bar < cumb[E]).astype(jnp.int32)
    row_off = cumb[:E] * T
    pos = jnp.take(row_off, eidx) + ranks
    tok_ids = jnp.broadcast_to(
        jnp.arange(N, dtype=jnp.int32)[:, None], (N, K))
    flat_pos = pos.reshape(-1)
    tok_of_slot = jnp.zeros((NP,), jnp.int32).at[flat_pos].set(
        tok_ids.reshape(-1))
    posT = pos.T.reshape(-1)

    w2b = W2.astype(jnp.bfloat16)
    tok_dep = tok_of_slot + (w2b[0, 0, 0] * 0).astype(jnp.int32)
    xg = _sc_gather(tok_dep, x_flat, NP, jnp.float32)

    yg = pl.pallas_call(
        _ffn_body,
        grid_spec=pltpu.PrefetchScalarGridSpec(
            num_scalar_prefetch=2,
            grid=(NB,),
            in_specs=[
                pl.BlockSpec((T, H), lambda b, be, va: (b, 0)),
                pl.BlockSpec((1, H, FF), lambda b, be, va: (be[b], 0, 0)),
                pl.BlockSpec((1, FF, H), lambda b, be, va: (be[b], 0, 0)),
                pl.BlockSpec((1, 1, FF), lambda b, be, va: (be[b], 0, 0)),
                pl.BlockSpec((1, 1, H), lambda b, be, va: (be[b], 0, 0)),
            ],
            out_specs=pl.BlockSpec((T, H), lambda b, be, va: (b, 0)),
        ),
        out_shape=jax.ShapeDtypeStruct((NP, H), jnp.float32),
    )(block_expert, valid, xg, W1, w2b,
      b1.reshape(E, 1, FF), b2.reshape(E, 1, H))

    yp = _sc_gather(posT, yg, NK, jnp.float32)

    NTC = N // 1024
    out = pl.pallas_call(
        _combine_body,
        grid=(NTC,),
        in_specs=[
            pl.BlockSpec((1024, H), lambda i: (i, 0)),
            pl.BlockSpec((1024, H), lambda i: (i + NTC, 0)),
            pl.BlockSpec((1024, K), lambda i: (i, 0)),
        ],
        out_specs=pl.BlockSpec((1024, H), lambda i: (i, 0)),
        out_shape=jax.ShapeDtypeStruct((N, H), jnp.float32),
    )(yp, yp, gates2)

    return out.reshape(x.shape), aux


# speedup vs baseline: 1.0030x; 1.0030x over previous
import jax
import jax.numpy as jnp
from jax import lax
from jax.experimental import pallas as pl
from jax.experimental.pallas import tpu as pltpu
from jax.experimental.pallas import tpu_sc as plsc

B, S, H = 4, 2048, 1024
E, K, FF = 8, 2, 4096
N = B * S
NK = N * K

T = 256
NB = NK // T + E
NP = NB * T

TB = 512
NT = N // TB

NEG_BIG = -1e30

NC = 2
NS = 16
NW = NC * NS
CH = 32


def _gating_body(x_ref, wg_ref, bg_ref, eidx_ref, gates_ref, ranks_ref,
                 colsum_ref, counts_ref, xcopy_ref, carry_ref):
    t = pl.program_id(0)

    @pl.when(t == 0)
    def _():
        carry_ref[...] = jnp.zeros_like(carry_ref)

    xv = x_ref[...]
    xcopy_ref[...] = xv
    logits = jnp.dot(xv, wg_ref[...],
                     preferred_element_type=jnp.float32) + bg_ref[...]
    m = jnp.max(logits, axis=1, keepdims=True)
    ex = jnp.exp(logits - m)
    probs = ex / jnp.sum(ex, axis=1, keepdims=True)
    colsum_ref[0] = jnp.sum(probs, axis=0, keepdims=True)

    iota = lax.broadcasted_iota(jnp.int32, (1, E), 1)
    p1 = jnp.max(probs, axis=1, keepdims=True)
    i1 = jnp.min(jnp.where(probs == p1, iota, E), axis=1, keepdims=True)
    masked = jnp.where(iota == i1, NEG_BIG, probs)
    p2 = jnp.max(masked, axis=1, keepdims=True)
    i2 = jnp.min(jnp.where(masked == p2, iota, E), axis=1, keepdims=True)
    dsum = p1 + p2
    gates_ref[...] = jnp.concatenate([p1 / dsum, p2 / dsum], axis=1)
    eidx_ref[...] = jnp.concatenate([i1, i2], axis=1)

    onehot = (iota == i1).astype(jnp.float32) + (iota == i2).astype(jnp.float32)
    r_io = lax.broadcasted_iota(jnp.int32, (TB, TB), 0)
    c_io = lax.broadcasted_iota(jnp.int32, (TB, TB), 1)
    tril = (r_io > c_io).astype(jnp.float32)
    prefix = jnp.dot(tril, onehot, preferred_element_type=jnp.float32)
    base = prefix + carry_ref[...]
    r1 = jnp.sum(jnp.where(iota == i1, base, 0.0), axis=1, keepdims=True)
    r2 = jnp.sum(jnp.where(iota == i2, base, 0.0), axis=1, keepdims=True)
    ranks_ref[...] = jnp.concatenate([r1, r2], axis=1).astype(jnp.int32)

    new_carry = carry_ref[...] + jnp.sum(onehot, axis=0, keepdims=True)
    carry_ref[...] = new_carry
    counts_ref[...] = new_carry


def _make_gather_body(rows_per_worker, src_rows, dtype):
    G = rows_per_worker // CH
    assert G % 2 == 0 and rows_per_worker % CH == 0

    def body(idx_hbm, src_hbm, out_hbm, idx_all, buf_a, buf_b,
             sg_a, sg_b, sw_a, sw_b):
        wid = lax.axis_index("s") * NC + lax.axis_index("c")
        base = wid * rows_per_worker
        pltpu.sync_copy(idx_hbm.at[pl.ds(base, rows_per_worker)], idx_all)

        def gather(g, buf, sem):
            return pltpu.async_copy(
                src_hbm.at[idx_all.at[pl.ds(g * CH, CH)]], buf, sem)

        def write(g, buf, sem):
            return pltpu.async_copy(
                buf, out_hbm.at[pl.ds(base + g * CH, CH)], sem)

        def wait_gather(buf, sem):
            pltpu.make_async_copy(src_hbm.at[pl.ds(0, CH)], buf, sem).wait()

        def wait_write(g, buf, sem):
            pltpu.make_async_copy(
                buf, out_hbm.at[pl.ds(base + g * CH, CH)], sem).wait()

        gather(0, buf_a, sg_a)

        def step(j, carry):
            g0 = 2 * j
            wait_gather(buf_a, sg_a)
            gather(g0 + 1, buf_b, sg_b)
            write(g0, buf_a, sw_a)
            wait_gather(buf_b, sg_b)
            wait_write(g0, buf_a, sw_a)
            g_next = jnp.minimum(g0 + 2, G - 1)
            gather(g_next, buf_a, sg_a)
            write(g0 + 1, buf_b, sw_b)
            wait_write(g0 + 1, buf_b, sw_b)
            return carry

        lax.fori_loop(0, G // 2, step, 0)
        wait_gather(buf_a, sg_a)

    scratch = [
        pltpu.VMEM((rows_per_worker,), jnp.int32),
        pltpu.VMEM((CH, H), dtype),
        pltpu.VMEM((CH, H), dtype),
        pltpu.SemaphoreType.DMA,
        pltpu.SemaphoreType.DMA,
        pltpu.SemaphoreType.DMA,
        pltpu.SemaphoreType.DMA,
    ]
    return body, scratch


def _sc_gather(idx, src, n_rows, dtype):
    body, scratch = _make_gather_body(n_rows // NW, src.shape[0], dtype)
    return pl.kernel(
        body,
        out_type=jax.ShapeDtypeStruct((n_rows, H), dtype),
        mesh=plsc.VectorSubcoreMesh(
            core_axis_name="c", subcore_axis_name="s",
            num_cores=NC, num_subcores=NS),
        scratch_types=scratch,
    )(idx, src)


def _ffn_body(be_ref, valid_ref, xg_ref, w1_ref, w2_ref, b1_ref, b2_ref,
              out_ref):
    b = pl.program_id(0)

    @pl.when(valid_ref[b] == 1)
    def _():
        h = jnp.dot(xg_ref[...], w1_ref[0],
                    preferred_element_type=jnp.float32)
        h = jnp.maximum(h + b1_ref[0, 0], 0.0).astype(jnp.bfloat16)
        y = jnp.dot(h, w2_ref[0], preferred_element_type=jnp.float32)
        out_ref[...] = y + b2_ref[0]

    @pl.when(valid_ref[b] == 0)
    def _():
        out_ref[...] = jnp.zeros_like(out_ref)


def _combine_body(a_ref, b_ref, g_ref, o_ref):
    g = g_ref[...]
    o_ref[...] = g[:, 0:1] * a_ref[...] + g[:, 1:2] * b_ref[...]


def kernel(x, Wg, bg, W1, b1, W2, b2):
    x_flat = x.reshape(-1, H)

    eidx, gates2, ranks, colsum, counts, xcopy = pl.pallas_call(
        _gating_body,
        grid=(NT,),
        in_specs=[
            pl.BlockSpec((TB, H), lambda t: (t, 0)),
            pl.BlockSpec((H, E), lambda t: (0, 0)),
            pl.BlockSpec((1, E), lambda t: (0, 0)),
        ],
        out_specs=[
            pl.BlockSpec((TB, K), lambda t: (t, 0)),
            pl.BlockSpec((TB, K), lambda t: (t, 0)),
            pl.BlockSpec((TB, K), lambda t: (t, 0)),
            pl.BlockSpec((1, 1, E), lambda t: (t, 0, 0)),
            pl.BlockSpec((1, E), lambda t: (0, 0)),
            pl.BlockSpec((TB, H), lambda t: (t, 0)),
        ],
        out_shape=[
            jax.ShapeDtypeStruct((N, K), jnp.int32),
            jax.ShapeDtypeStruct((N, K), jnp.float32),
            jax.ShapeDtypeStruct((N, K), jnp.int32),
            jax.ShapeDtypeStruct((NT, 1, E), jnp.float32),
            jax.ShapeDtypeStruct((1, E), jnp.float32),
            jax.ShapeDtypeStruct((N, H), jnp.float32),
        ],
        scratch_shapes=[pltpu.VMEM((1, E), jnp.float32)],
    )(x_flat, Wg, bg.reshape(1, E))

    cs = jnp.sum(colsum, axis=(0, 1))
    aux = E * jnp.sum((cs / jnp.sum(cs)) * (cs / N))

    counts_i = counts[0].astype(jnp.int32)
    blocks_e = (counts_i + (T - 1)) // T
    cumb = jnp.concatenate(
        [jnp.zeros((1,), jnp.int32), jnp.cumsum(blocks_e, dtype=jnp.int32)])
    bar = jnp.arange(NB, dtype=jnp.int32)
    block_expert = jnp.sum(
        (bar[:, None] >= cumb[None, 1:E]).astype(jnp.int32), axis=1)
    valid = (bar < cumb[E]).astype(jnp.int32)
    row_off = cumb[:E] * T
    pos = jnp.take(row_off, eidx) + ranks
    tok_ids = jnp.broadcast_to(
        jnp.arange(N, dtype=jnp.int32)[:, None], (N, K))
    flat_pos = pos.reshape(-1)
    tok_of_slot = jnp.zeros((NP,), jnp.int32).at[flat_pos].set(
        tok_ids.reshape(-1))
    posT = pos.T.reshape(-1)

    w2b = W2.astype(jnp.bfloat16)
    xg = _sc_gather(tok_of_slot, xcopy, NP, jnp.float32)

    yg = pl.pallas_call(
        _ffn_body,
        grid_spec=pltpu.PrefetchScalarGridSpec(
            num_scalar_prefetch=2,
            grid=(NB,),
            in_specs=[
                pl.BlockSpec((T, H), lambda b, be, va: (b, 0)),
                pl.BlockSpec((1, H, FF), lambda b, be, va: (be[b], 0, 0)),
                pl.BlockSpec((1, FF, H), lambda b, be, va: (be[b], 0, 0)),
                pl.BlockSpec((1, 1, FF), lambda b, be, va: (be[b], 0, 0)),
                pl.BlockSpec((1, 1, H), lambda b, be, va: (be[b], 0, 0)),
            ],
            out_specs=pl.BlockSpec((T, H), lambda b, be, va: (b, 0)),
        ),
        out_shape=jax.ShapeDtypeStruct((NP, H), jnp.float32),
    )(block_expert, valid, xg, W1, w2b,
      b1.reshape(E, 1, FF), b2.reshape(E, 1, H))

    yp = _sc_gather(posT, yg, NK, jnp.float32)

    NTC = N // 1024
    out = pl.pallas_call(
        _combine_body,
        grid=(NTC,),
        in_specs=[
            pl.BlockSpec((1024, H), lambda i: (i, 0)),
            pl.BlockSpec((1024, H), lambda i: (i + NTC, 0)),
            pl.BlockSpec((1024, K), lambda i: (i, 0)),
        ],
        out_specs=pl.BlockSpec((1024, H), lambda i: (i, 0)),
        out_shape=jax.ShapeDtypeStruct((N, H), jnp.float32),
    )(yp, yp, gates2)

    return out.reshape(x.shape), aux


# speedup vs baseline: 1.1758x; 1.1723x over previous
import jax
import jax.numpy as jnp
from jax import lax
from jax.experimental import pallas as pl
from jax.experimental.pallas import tpu as pltpu
from jax.experimental.pallas import tpu_sc as plsc

B, S, H = 4, 2048, 1024
E, K, FF = 8, 2, 4096
N = B * S
NK = N * K

T = 256
NB = NK // T + E
NP = NB * T

TB = 512
NT = N // TB

NEG_BIG = -1e30

NC = 2
NS = 16
NW = NC * NS
CH = 32


def _gating_body(x_ref, wg_ref, bg_ref, eidx_ref, gates_ref, ranks_ref,
                 colsum_ref, counts_ref, xcopy_ref, carry_ref):
    t = pl.program_id(0)

    @pl.when(t == 0)
    def _():
        carry_ref[...] = jnp.zeros_like(carry_ref)

    xv = x_ref[...]
    xcopy_ref[...] = xv
    logits = jnp.dot(xv, wg_ref[...],
                     preferred_element_type=jnp.float32) + bg_ref[...]
    m = jnp.max(logits, axis=1, keepdims=True)
    ex = jnp.exp(logits - m)
    probs = ex / jnp.sum(ex, axis=1, keepdims=True)
    colsum_ref[0] = jnp.sum(probs, axis=0, keepdims=True)

    iota = lax.broadcasted_iota(jnp.int32, (1, E), 1)
    p1 = jnp.max(probs, axis=1, keepdims=True)
    i1 = jnp.min(jnp.where(probs == p1, iota, E), axis=1, keepdims=True)
    masked = jnp.where(iota == i1, NEG_BIG, probs)
    p2 = jnp.max(masked, axis=1, keepdims=True)
    i2 = jnp.min(jnp.where(masked == p2, iota, E), axis=1, keepdims=True)
    dsum = p1 + p2
    gates_ref[...] = jnp.concatenate([p1 / dsum, p2 / dsum], axis=1)
    eidx_ref[...] = jnp.concatenate([i1, i2], axis=1)

    onehot = (iota == i1).astype(jnp.float32) + (iota == i2).astype(jnp.float32)
    r_io = lax.broadcasted_iota(jnp.int32, (TB, TB), 0)
    c_io = lax.broadcasted_iota(jnp.int32, (TB, TB), 1)
    tril = (r_io > c_io).astype(jnp.float32)
    prefix = jnp.dot(tril, onehot, preferred_element_type=jnp.float32)
    base = prefix + carry_ref[...]
    r1 = jnp.sum(jnp.where(iota == i1, base, 0.0), axis=1, keepdims=True)
    r2 = jnp.sum(jnp.where(iota == i2, base, 0.0), axis=1, keepdims=True)
    ranks_ref[...] = jnp.concatenate([r1, r2], axis=1).astype(jnp.int32)

    new_carry = carry_ref[...] + jnp.sum(onehot, axis=0, keepdims=True)
    carry_ref[...] = new_carry
    counts_ref[...] = new_carry


def _make_gather_body(rows_per_worker, src_rows, dtype):
    G = rows_per_worker // CH
    assert G % 2 == 0 and rows_per_worker % CH == 0

    def body(idx_hbm, src_hbm, out_hbm, idx_all, buf_a, buf_b,
             sg_a, sg_b, sw_a, sw_b):
        wid = lax.axis_index("s") * NC + lax.axis_index("c")
        base = wid * rows_per_worker
        pltpu.sync_copy(idx_hbm.at[pl.ds(base, rows_per_worker)], idx_all)

        def gather(g, buf, sem):
            return pltpu.async_copy(
                src_hbm.at[idx_all.at[pl.ds(g * CH, CH)]], buf, sem)

        def write(g, buf, sem):
            return pltpu.async_copy(
                buf, out_hbm.at[pl.ds(base + g * CH, CH)], sem)

        def wait_gather(buf, sem):
            pltpu.make_async_copy(src_hbm.at[pl.ds(0, CH)], buf, sem).wait()

        def wait_write(g, buf, sem):
            pltpu.make_async_copy(
                buf, out_hbm.at[pl.ds(base + g * CH, CH)], sem).wait()

        gather(0, buf_a, sg_a)

        def step(j, carry):
            g0 = 2 * j
            wait_gather(buf_a, sg_a)
            gather(g0 + 1, buf_b, sg_b)
            write(g0, buf_a, sw_a)
            wait_gather(buf_b, sg_b)
            wait_write(g0, buf_a, sw_a)
            g_next = jnp.minimum(g0 + 2, G - 1)
            gather(g_next, buf_a, sg_a)
            write(g0 + 1, buf_b, sw_b)
            wait_write(g0 + 1, buf_b, sw_b)
            return carry

        lax.fori_loop(0, G // 2, step, 0)
        wait_gather(buf_a, sg_a)

    scratch = [
        pltpu.VMEM((rows_per_worker,), jnp.int32),
        pltpu.VMEM((CH, H), dtype),
        pltpu.VMEM((CH, H), dtype),
        pltpu.SemaphoreType.DMA,
        pltpu.SemaphoreType.DMA,
        pltpu.SemaphoreType.DMA,
        pltpu.SemaphoreType.DMA,
    ]
    return body, scratch


def _sc_gather(idx, src, n_rows, dtype):
    body, scratch = _make_gather_body(n_rows // NW, src.shape[0], dtype)
    return pl.kernel(
        body,
        out_type=jax.ShapeDtypeStruct((n_rows, H), dtype),
        mesh=plsc.VectorSubcoreMesh(
            core_axis_name="c", subcore_axis_name="s",
            num_cores=NC, num_subcores=NS),
        scratch_types=scratch,
    )(idx, src)


def _ffn_body(be_ref, valid_ref, xg_ref, w1_ref, w2_ref, b1_ref, b2_ref,
              out_ref):
    b = pl.program_id(0)

    @pl.when(valid_ref[b] == 1)
    def _():
        h = jnp.dot(xg_ref[...], w1_ref[0],
                    preferred_element_type=jnp.float32)
        h = jnp.maximum(h + b1_ref[0, 0], 0.0).astype(jnp.bfloat16)
        y = jnp.dot(h, w2_ref[0], preferred_element_type=jnp.float32)
        out_ref[...] = y + b2_ref[0]

    @pl.when(valid_ref[b] == 0)
    def _():
        out_ref[...] = jnp.zeros_like(out_ref)


def _combine_body(a_ref, b_ref, g_ref, o_ref):
    g = g_ref[...]
    o_ref[...] = g[:, 0:1] * a_ref[...] + g[:, 1:2] * b_ref[...]


def kernel(x, Wg, bg, W1, b1, W2, b2):
    x_flat = x.reshape(-1, H)

    eidx, gates2, ranks, colsum, counts, xcopy = pl.pallas_call(
        _gating_body,
        grid=(NT,),
        in_specs=[
            pl.BlockSpec((TB, H), lambda t: (t, 0)),
            pl.BlockSpec((H, E), lambda t: (0, 0)),
            pl.BlockSpec((1, E), lambda t: (0, 0)),
        ],
        out_specs=[
            pl.BlockSpec((TB, K), lambda t: (t, 0)),
            pl.BlockSpec((TB, K), lambda t: (t, 0)),
            pl.BlockSpec((TB, K), lambda t: (t, 0)),
            pl.BlockSpec((1, 1, E), lambda t: (t, 0, 0)),
            pl.BlockSpec((1, E), lambda t: (0, 0)),
            pl.BlockSpec((TB, H), lambda t: (t, 0)),
        ],
        out_shape=[
            jax.ShapeDtypeStruct((N, K), jnp.int32),
            jax.ShapeDtypeStruct((N, K), jnp.float32),
            jax.ShapeDtypeStruct((N, K), jnp.int32),
            jax.ShapeDtypeStruct((NT, 1, E), jnp.float32),
            jax.ShapeDtypeStruct((1, E), jnp.float32),
            jax.ShapeDtypeStruct((N, H), jnp.float32),
        ],
        scratch_shapes=[pltpu.VMEM((1, E), jnp.float32)],
    )(x_flat, Wg, bg.reshape(1, E))

    cs = jnp.sum(colsum, axis=(0, 1))
    aux = E * jnp.sum((cs / jnp.sum(cs)) * (cs / N))

    counts_i = counts[0].astype(jnp.int32)
    blocks_e = (counts_i + (T - 1)) // T
    cumb = jnp.concatenate(
        [jnp.zeros((1,), jnp.int32), jnp.cumsum(blocks_e, dtype=jnp.int32)])
    bar = jnp.arange(NB, dtype=jnp.int32)
    block_expert = jnp.sum(
        (bar[:, None] >= cumb[None, 1:E]).astype(jnp.int32), axis=1)
    valid = (bar < cumb[E]).astype(jnp.int32)
    row_off = cumb[:E] * T
    pos = jnp.take(row_off, eidx) + ranks
    tok_ids = jnp.broadcast_to(
        jnp.arange(N, dtype=jnp.int32)[:, None], (N, K))
    flat_pos = pos.reshape(-1)
    tok_of_slot = (jnp.arange(NP, dtype=jnp.int32) % N).at[flat_pos].set(
        tok_ids.reshape(-1))
    posT = pos.T.reshape(-1)

    w2b = W2.astype(jnp.bfloat16)
    xg = _sc_gather(tok_of_slot, xcopy, NP, jnp.float32)

    yg = pl.pallas_call(
        _ffn_body,
        grid_spec=pltpu.PrefetchScalarGridSpec(
            num_scalar_prefetch=2,
            grid=(NB,),
            in_specs=[
                pl.BlockSpec((T, H), lambda b, be, va: (b, 0)),
                pl.BlockSpec((1, H, FF), lambda b, be, va: (be[b], 0, 0)),
                pl.BlockSpec((1, FF, H), lambda b, be, va: (be[b], 0, 0)),
                pl.BlockSpec((1, 1, FF), lambda b, be, va: (be[b], 0, 0)),
                pl.BlockSpec((1, 1, H), lambda b, be, va: (be[b], 0, 0)),
            ],
            out_specs=pl.BlockSpec((T, H), lambda b, be, va: (b, 0)),
        ),
        out_shape=jax.ShapeDtypeStruct((NP, H), jnp.float32),
    )(block_expert, valid, xg, W1, w2b,
      b1.reshape(E, 1, FF), b2.reshape(E, 1, H))

    yp = _sc_gather(posT, yg, NK, jnp.float32)

    NTC = N // 1024
    out = pl.pallas_call(
        _combine_body,
        grid=(NTC,),
        in_specs=[
            pl.BlockSpec((1024, H), lambda i: (i, 0)),
            pl.BlockSpec((1024, H), lambda i: (i + NTC, 0)),
            pl.BlockSpec((1024, K), lambda i: (i, 0)),
        ],
        out_specs=pl.BlockSpec((1024, H), lambda i: (i, 0)),
        out_shape=jax.ShapeDtypeStruct((N, H), jnp.float32),
    )(yp, yp, gates2)

    return out.reshape(x.shape), aux


# speedup vs baseline: 1.3289x; 1.1302x over previous
import jax
import jax.numpy as jnp
from jax import lax
from jax.experimental import pallas as pl
from jax.experimental.pallas import tpu as pltpu
from jax.experimental.pallas import tpu_sc as plsc

B, S, H = 4, 2048, 1024
E, K, FF = 8, 2, 4096
N = B * S
NK = N * K

T = 256
NB = NK // T + E
NP = NB * T

TB = 512
NT = N // TB

NEG_BIG = -1e30

NC = 2
NS = 16
NW = NC * NS
CH = 32
TW = N // NW


def _gating_body(x_ref, wg_ref, bg_ref, eidx_ref, gates_ref, ranks_ref,
                 colsum_ref, counts_ref, xcopy_ref, carry_ref):
    t = pl.program_id(0)

    @pl.when(t == 0)
    def _():
        carry_ref[...] = jnp.zeros_like(carry_ref)

    xv = x_ref[...]
    xcopy_ref[...] = xv
    logits = jnp.dot(xv, wg_ref[...],
                     preferred_element_type=jnp.float32) + bg_ref[...]
    m = jnp.max(logits, axis=1, keepdims=True)
    ex = jnp.exp(logits - m)
    probs = ex / jnp.sum(ex, axis=1, keepdims=True)
    colsum_ref[0] = jnp.sum(probs, axis=0, keepdims=True)

    iota = lax.broadcasted_iota(jnp.int32, (1, E), 1)
    p1 = jnp.max(probs, axis=1, keepdims=True)
    i1 = jnp.min(jnp.where(probs == p1, iota, E), axis=1, keepdims=True)
    masked = jnp.where(iota == i1, NEG_BIG, probs)
    p2 = jnp.max(masked, axis=1, keepdims=True)
    i2 = jnp.min(jnp.where(masked == p2, iota, E), axis=1, keepdims=True)
    dsum = p1 + p2
    gates_ref[...] = jnp.concatenate([p1 / dsum, p2 / dsum], axis=1)
    eidx_ref[...] = jnp.concatenate([i1, i2], axis=1)

    onehot = (iota == i1).astype(jnp.float32) + (iota == i2).astype(jnp.float32)
    r_io = lax.broadcasted_iota(jnp.int32, (TB, TB), 0)
    c_io = lax.broadcasted_iota(jnp.int32, (TB, TB), 1)
    tril = (r_io > c_io).astype(jnp.float32)
    prefix = jnp.dot(tril, onehot, preferred_element_type=jnp.float32)
    base = prefix + carry_ref[...]
    r1 = jnp.sum(jnp.where(iota == i1, base, 0.0), axis=1, keepdims=True)
    r2 = jnp.sum(jnp.where(iota == i2, base, 0.0), axis=1, keepdims=True)
    ranks_ref[...] = jnp.concatenate([r1, r2], axis=1).astype(jnp.int32)

    new_carry = carry_ref[...] + jnp.sum(onehot, axis=0, keepdims=True)
    carry_ref[...] = new_carry
    counts_ref[...] = new_carry


CT = 32
G2 = (N // NW) // CT


def _dispatch_body(pos0_hbm, pos1_hbm, x_hbm, xg_hbm,
                   idx0_v, idx1_v, buf_a, buf_b, sr_a, sr_b, sw_a, sw_b):
    wid = lax.axis_index("s") * NC + lax.axis_index("c")
    tbase = wid * TW
    pltpu.sync_copy(pos0_hbm.at[wid], idx0_v)
    pltpu.sync_copy(pos1_hbm.at[wid], idx1_v)

    def read(c, buf, sem):
        pltpu.async_copy(x_hbm.at[pl.ds(tbase + c * CT, CT)], buf, sem)

    def wait_read(buf, sem):
        pltpu.make_async_copy(x_hbm.at[pl.ds(0, CT)], buf, sem).wait()

    def scatter(c, buf, sem):
        pltpu.async_copy(buf, xg_hbm.at[idx0_v.at[c]], sem)
        pltpu.async_copy(buf, xg_hbm.at[idx1_v.at[c]], sem)

    def wait_scatter(buf, sem):
        pltpu.make_async_copy(buf, xg_hbm.at[pl.ds(0, CT)], sem).wait()
        pltpu.make_async_copy(buf, xg_hbm.at[pl.ds(0, CT)], sem).wait()

    read(0, buf_a, sr_a)

    def step(j, carry):
        c0 = 2 * j
        wait_read(buf_a, sr_a)
        read(c0 + 1, buf_b, sr_b)
        scatter(c0, buf_a, sw_a)
        wait_read(buf_b, sr_b)
        wait_scatter(buf_a, sw_a)
        read(jnp.minimum(c0 + 2, G2 - 1), buf_a, sr_a)
        scatter(c0 + 1, buf_b, sw_b)
        wait_scatter(buf_b, sw_b)
        return carry

    lax.fori_loop(0, G2 // 2, step, 0)
    wait_read(buf_a, sr_a)


_DISPATCH_SCRATCH = [
    pltpu.VMEM((G2, CT), jnp.int32),
    pltpu.VMEM((G2, CT), jnp.int32),
    pltpu.VMEM((CT, H), jnp.float32),
    pltpu.VMEM((CT, H), jnp.float32),
    pltpu.SemaphoreType.DMA,
    pltpu.SemaphoreType.DMA,
    pltpu.SemaphoreType.DMA,
    pltpu.SemaphoreType.DMA,
]


def _make_gather_body(rows_per_worker, src_rows, dtype):
    G = rows_per_worker // CH
    assert G % 2 == 0 and rows_per_worker % CH == 0

    def body(idx_hbm, src_hbm, out_hbm, idx_all, buf_a, buf_b,
             sg_a, sg_b, sw_a, sw_b):
        wid = lax.axis_index("s") * NC + lax.axis_index("c")
        base = wid * rows_per_worker
        pltpu.sync_copy(idx_hbm.at[pl.ds(base, rows_per_worker)], idx_all)

        def gather(g, buf, sem):
            return pltpu.async_copy(
                src_hbm.at[idx_all.at[pl.ds(g * CH, CH)]], buf, sem)

        def write(g, buf, sem):
            return pltpu.async_copy(
                buf, out_hbm.at[pl.ds(base + g * CH, CH)], sem)

        def wait_gather(buf, sem):
            pltpu.make_async_copy(src_hbm.at[pl.ds(0, CH)], buf, sem).wait()

        def wait_write(g, buf, sem):
            pltpu.make_async_copy(
                buf, out_hbm.at[pl.ds(base + g * CH, CH)], sem).wait()

        gather(0, buf_a, sg_a)

        def step(j, carry):
            g0 = 2 * j
            wait_gather(buf_a, sg_a)
            gather(g0 + 1, buf_b, sg_b)
            write(g0, buf_a, sw_a)
            wait_gather(buf_b, sg_b)
            wait_write(g0, buf_a, sw_a)
            g_next = jnp.minimum(g0 + 2, G - 1)
            gather(g_next, buf_a, sg_a)
            write(g0 + 1, buf_b, sw_b)
            wait_write(g0 + 1, buf_b, sw_b)
            return carry

        lax.fori_loop(0, G // 2, step, 0)
        wait_gather(buf_a, sg_a)

    scratch = [
        pltpu.VMEM((rows_per_worker,), jnp.int32),
        pltpu.VMEM((CH, H), dtype),
        pltpu.VMEM((CH, H), dtype),
        pltpu.SemaphoreType.DMA,
        pltpu.SemaphoreType.DMA,
        pltpu.SemaphoreType.DMA,
        pltpu.SemaphoreType.DMA,
    ]
    return body, scratch


def _sc_dispatch(pos0r, pos1r, xsrc):
    return pl.kernel(
        _dispatch_body,
        out_type=jax.ShapeDtypeStruct((NP, H), jnp.float32),
        mesh=plsc.VectorSubcoreMesh(
            core_axis_name="c", subcore_axis_name="s",
            num_cores=NC, num_subcores=NS),
        scratch_types=_DISPATCH_SCRATCH,
    )(pos0r, pos1r, xsrc)


def _sc_gather(idx, src, n_rows, dtype):
    body, scratch = _make_gather_body(n_rows // NW, src.shape[0], dtype)
    return pl.kernel(
        body,
        out_type=jax.ShapeDtypeStruct((n_rows, H), dtype),
        mesh=plsc.VectorSubcoreMesh(
            core_axis_name="c", subcore_axis_name="s",
            num_cores=NC, num_subcores=NS),
        scratch_types=scratch,
    )(idx, src)


def _ffn_body(be_ref, valid_ref, xg_ref, w1_ref, w2_ref, b1_ref, b2_ref,
              out_ref):
    b = pl.program_id(0)

    @pl.when(valid_ref[b] == 1)
    def _():
        h = jnp.dot(xg_ref[...], w1_ref[0],
                    preferred_element_type=jnp.float32)
        h = jnp.maximum(h + b1_ref[0, 0], 0.0).astype(jnp.bfloat16)
        y = jnp.dot(h, w2_ref[0], preferred_element_type=jnp.float32)
        out_ref[...] = y + b2_ref[0]

    @pl.when(valid_ref[b] == 0)
    def _():
        out_ref[...] = jnp.zeros_like(out_ref)


def _combine_body(a_ref, b_ref, g_ref, o_ref):
    g = g_ref[...]
    o_ref[...] = g[:, 0:1] * a_ref[...] + g[:, 1:2] * b_ref[...]


def kernel(x, Wg, bg, W1, b1, W2, b2):
    x_flat = x.reshape(-1, H)

    eidx, gates2, ranks, colsum, counts, xcopy = pl.pallas_call(
        _gating_body,
        grid=(NT,),
        in_specs=[
            pl.BlockSpec((TB, H), lambda t: (t, 0)),
            pl.BlockSpec((H, E), lambda t: (0, 0)),
            pl.BlockSpec((1, E), lambda t: (0, 0)),
        ],
        out_specs=[
            pl.BlockSpec((TB, K), lambda t: (t, 0)),
            pl.BlockSpec((TB, K), lambda t: (t, 0)),
            pl.BlockSpec((TB, K), lambda t: (t, 0)),
            pl.BlockSpec((1, 1, E), lambda t: (t, 0, 0)),
            pl.BlockSpec((1, E), lambda t: (0, 0)),
            pl.BlockSpec((TB, H), lambda t: (t, 0)),
        ],
        out_shape=[
            jax.ShapeDtypeStruct((N, K), jnp.int32),
            jax.ShapeDtypeStruct((N, K), jnp.float32),
            jax.ShapeDtypeStruct((N, K), jnp.int32),
            jax.ShapeDtypeStruct((NT, 1, E), jnp.float32),
            jax.ShapeDtypeStruct((1, E), jnp.float32),
            jax.ShapeDtypeStruct((N, H), jnp.float32),
        ],
        scratch_shapes=[pltpu.VMEM((1, E), jnp.float32)],
    )(x_flat, Wg, bg.reshape(1, E))

    cs = jnp.sum(colsum, axis=(0, 1))
    aux = E * jnp.sum((cs / jnp.sum(cs)) * (cs / N))

    counts_i = counts[0].astype(jnp.int32)
    blocks_e = (counts_i + (T - 1)) // T
    cumb = jnp.concatenate(
        [jnp.zeros((1,), jnp.int32), jnp.cumsum(blocks_e, dtype=jnp.int32)])
    bar = jnp.arange(NB, dtype=jnp.int32)
    block_expert = jnp.sum(
        (bar[:, None] >= cumb[None, 1:E]).astype(jnp.int32), axis=1)
    valid = (bar < cumb[E]).astype(jnp.int32)
    row_off = cumb[:E] * T
    pos = jnp.take(row_off, eidx) + ranks
    posT = pos.T.reshape(-1)
    pos0r = posT[:N].reshape(NW, G2, CT)
    pos1r = posT[N:].reshape(NW, G2, CT)

    w2b = W2.astype(jnp.bfloat16)
    xg = _sc_dispatch(pos0r, pos1r, xcopy)

    yg = pl.pallas_call(
        _ffn_body,
        grid_spec=pltpu.PrefetchScalarGridSpec(
            num_scalar_prefetch=2,
            grid=(NB,),
            in_specs=[
                pl.BlockSpec((T, H), lambda b, be, va: (b, 0)),
                pl.BlockSpec((1, H, FF), lambda b, be, va: (be[b], 0, 0)),
                pl.BlockSpec((1, FF, H), lambda b, be, va: (be[b], 0, 0)),
                pl.BlockSpec((1, 1, FF), lambda b, be, va: (be[b], 0, 0)),
                pl.BlockSpec((1, 1, H), lambda b, be, va: (be[b], 0, 0)),
            ],
            out_specs=pl.BlockSpec((T, H), lambda b, be, va: (b, 0)),
        ),
        out_shape=jax.ShapeDtypeStruct((NP, H), jnp.float32),
    )(block_expert, valid, xg, W1, w2b,
      b1.reshape(E, 1, FF), b2.reshape(E, 1, H))

    yp = _sc_gather(posT, yg, NK, jnp.float32)

    NTC = N // 1024
    out = pl.pallas_call(
        _combine_body,
        grid=(NTC,),
        in_specs=[
            pl.BlockSpec((1024, H), lambda i: (i, 0)),
            pl.BlockSpec((1024, H), lambda i: (i + NTC, 0)),
            pl.BlockSpec((1024, K), lambda i: (i, 0)),
        ],
        out_specs=pl.BlockSpec((1024, H), lambda i: (i, 0)),
        out_shape=jax.ShapeDtypeStruct((N, H), jnp.float32),
    )(yp, yp, gates2)

    return out.reshape(x.shape), aux
